# baseline (speedup 1.0000x reference)
import functools

import jax
import jax.numpy as jnp
from jax.experimental import pallas as pl
from jax.experimental.pallas import tpu as pltpu


def _pool_body(adj_ref, x_ref, xs_ref, o_ref):
    a = (adj_ref[...] == 1).astype(jnp.bfloat16)
    p = jnp.dot(a, x_ref[...], preferred_element_type=jnp.float32)
    o_ref[...] = xs_ref[...].astype(jnp.float32) + p


@functools.partial(jax.jit, static_argnames=("bm",))
def _pool(x, adj, bm):
    n, d = x.shape
    xb = x.astype(jnp.bfloat16)
    return pl.pallas_call(
        _pool_body,
        grid=(n // bm,),
        in_specs=[
            pl.BlockSpec((bm, n), lambda i: (i, 0)),
            pl.BlockSpec((n, d), lambda i: (0, 0)),
            pl.BlockSpec((bm, d), lambda i: (i, 0)),
        ],
        out_specs=pl.BlockSpec((bm, d), lambda i: (i, 0)),
        out_shape=jax.ShapeDtypeStruct((n, d), jnp.float32),
        compiler_params=pltpu.CompilerParams(
            dimension_semantics=("parallel",),
        ),
    )(adj, xb, xb)


def kernel(x, adj):
    return _pool(x, adj, bm=200)
